# Initial kernel scaffold; baseline (speedup 1.0000x reference)
import jax
import jax.numpy as jnp
from jax.experimental import pallas as pl


def kernel(positions, atomic_numbers, pairs, shifts, W_embed, radial_w1, radial_b1, radial_w2, W_msg, W_read):
    raise NotImplementedError("write your pallas kernel here")


# speedup vs baseline: 5.9879x; 5.9879x over previous
import functools

import jax
import jax.numpy as jnp
from jax import lax
from jax.experimental import pallas as pl
from jax.experimental.pallas import tpu as pltpu
from jax.experimental.pallas import tpu_sc as plsc

N = 10000
E = 320000
NUM_EL = 4
H = 128
NB = 8
L = 2
R_MAX = 5.0
LENGTH_CONV = 10.0

NC = 2
NS = 16
NW = NC * NS
EPW = E // NW
CHUNK = 80
NCHUNK = EPW // CHUNK
RPW = 624

_SC_MESH = plsc.VectorSubcoreMesh(core_axis_name="c", subcore_axis_name="s")


def _d2_body(src_hbm, dst_hbm, sx_hbm, sy_hbm, sz_hbm,
             px_hbm, py_hbm, pz_hbm, d2_hbm,
             px_v, py_v, pz_v, src_v, dst_v, sx_v, sy_v, sz_v, d2_v):
    c = lax.axis_index("c")
    s = lax.axis_index("s")
    wid = s * NC + c
    base = wid * EPW
    pltpu.sync_copy(px_hbm, px_v)
    pltpu.sync_copy(py_hbm, py_v)
    pltpu.sync_copy(pz_hbm, pz_v)
    pltpu.sync_copy(src_hbm.at[pl.ds(base, EPW)], src_v)
    pltpu.sync_copy(dst_hbm.at[pl.ds(base, EPW)], dst_v)
    pltpu.sync_copy(sx_hbm.at[pl.ds(base, EPW)], sx_v)
    pltpu.sync_copy(sy_hbm.at[pl.ds(base, EPW)], sy_v)
    pltpu.sync_copy(sz_hbm.at[pl.ds(base, EPW)], sz_v)

    def step(i, carry):
        sl = pl.ds(i * 16, 16)
        vs = src_v[sl]
        vd = dst_v[sl]
        dx = plsc.load_gather(px_v, [vd]) - plsc.load_gather(px_v, [vs]) - sx_v[sl]
        dy = plsc.load_gather(py_v, [vd]) - plsc.load_gather(py_v, [vs]) - sy_v[sl]
        dz = plsc.load_gather(pz_v, [vd]) - plsc.load_gather(pz_v, [vs]) - sz_v[sl]
        d2_v[sl] = (dx * dx + dy * dy + dz * dz) * (LENGTH_CONV * LENGTH_CONV)
        return carry

    lax.fori_loop(0, EPW // 16, step, 0)
    pltpu.sync_copy(d2_v, d2_hbm.at[pl.ds(base, EPW)])


@functools.partial(
    pl.kernel,
    out_type=jax.ShapeDtypeStruct((E,), jnp.float32),
    mesh=_SC_MESH,
    scratch_types=[
        pltpu.VMEM((N,), jnp.float32),
        pltpu.VMEM((N,), jnp.float32),
        pltpu.VMEM((N,), jnp.float32),
        pltpu.VMEM((EPW,), jnp.int32),
        pltpu.VMEM((EPW,), jnp.int32),
        pltpu.VMEM((EPW,), jnp.float32),
        pltpu.VMEM((EPW,), jnp.float32),
        pltpu.VMEM((EPW,), jnp.float32),
        pltpu.VMEM((EPW,), jnp.float32),
    ],
    compiler_params=pltpu.CompilerParams(needs_layout_passes=False),
)
def _d2_kernel(*refs):
    _d2_body(*refs)


def _layer_body(h_hbm, rt_hbm, src_hbm, dst_hbm, zero_hbm, out_hbm,
                src_v, dst_v, rt_v, hs_v, hd_v, agg_sh, sem):
    c = lax.axis_index("c")
    s = lax.axis_index("s")
    wid = s * NC + c
    pltpu.sync_copy(zero_hbm.at[pl.ds(s * RPW, RPW)], agg_sh.at[pl.ds(s * RPW, RPW)])

    @pl.when(s == NS - 1)
    def _():
        pltpu.sync_copy(zero_hbm.at[pl.ds(NS * RPW, N - NS * RPW)],
                        agg_sh.at[pl.ds(NS * RPW, N - NS * RPW)])

    plsc.subcore_barrier()

    base0 = wid * EPW

    def chunk(i, carry):
        base = base0 + i * CHUNK
        pltpu.sync_copy(src_hbm.at[pl.ds(base, CHUNK)], src_v)
        pltpu.sync_copy(dst_hbm.at[pl.ds(base, CHUNK)], dst_v)
        pltpu.sync_copy(rt_hbm.at[pl.ds(base, CHUNK)], rt_v)
        cp1 = pltpu.async_copy(h_hbm.at[src_v], hs_v, sem)
        cp2 = pltpu.async_copy(h_hbm.at[dst_v], hd_v, sem)
        cp1.wait()
        cp2.wait()

        def row(r, rcarry):
            for cc in range(H // 16):
                fsl = pl.ds(cc * 16, 16)
                rt = rt_v[r, fsl]
                hs_v[r, fsl] = hs_v[r, fsl] * rt
                hd_v[r, fsl] = hd_v[r, fsl] * rt
            return rcarry

        lax.fori_loop(0, CHUNK, row, 0)
        pltpu.sync_copy(hs_v, agg_sh.at[dst_v], add=True)
        pltpu.sync_copy(hd_v, agg_sh.at[src_v], add=True)
        return carry

    lax.fori_loop(0, NCHUNK, chunk, 0)
    plsc.subcore_barrier()
    pltpu.sync_copy(agg_sh.at[pl.ds(s * RPW, RPW)], out_hbm.at[c, pl.ds(s * RPW, RPW)])

    @pl.when(s == NS - 1)
    def _():
        pltpu.sync_copy(agg_sh.at[pl.ds(NS * RPW, N - NS * RPW)],
                        out_hbm.at[c, pl.ds(NS * RPW, N - NS * RPW)])


@functools.partial(
    pl.kernel,
    out_type=jax.ShapeDtypeStruct((NC, N, H), jnp.float32),
    mesh=_SC_MESH,
    scratch_types=[
        pltpu.VMEM((CHUNK,), jnp.int32),
        pltpu.VMEM((CHUNK,), jnp.int32),
        pltpu.VMEM((CHUNK, H), jnp.float32),
        pltpu.VMEM((CHUNK, H), jnp.float32),
        pltpu.VMEM((CHUNK, H), jnp.float32),
        pltpu.VMEM_SHARED((N, H), jnp.float32),
        pltpu.SemaphoreType.DMA,
    ],
    compiler_params=pltpu.CompilerParams(needs_layout_passes=False),
)
def _layer_kernel(*refs):
    _layer_body(*refs)


_EB = 2000


def _radial_body(d2_ref, w10_ref, b10_ref, w20_ref, w11_ref, b11_ref, w21_ref,
                 rt0_ref, rt1_ref):
    d2 = d2_ref[...]
    r = jnp.sqrt(d2 + 1e-12)
    u = r * (1.0 / R_MAX)
    f = 1.0 - 28.0 * u**6 + 48.0 * u**7 - 21.0 * u**8
    cut = jnp.where(u < 1.0, f, 0.0)
    pref = jnp.sqrt(2.0 / R_MAX) * cut / r
    narr = (lax.broadcasted_iota(jnp.int32, (_EB, NB), 1) + 1).astype(jnp.float32)
    rbf = jnp.sin(narr * (jnp.pi / R_MAX) * r) * pref
    phi0 = jax.nn.silu(jnp.dot(rbf, w10_ref[...], preferred_element_type=jnp.float32) + b10_ref[...])
    rt0_ref[...] = jnp.dot(phi0, w20_ref[...], preferred_element_type=jnp.float32)
    phi1 = jax.nn.silu(jnp.dot(rbf, w11_ref[...], preferred_element_type=jnp.float32) + b11_ref[...])
    rt1_ref[...] = jnp.dot(phi1, w21_ref[...], preferred_element_type=jnp.float32)


def _radial(d2, radial_w1, radial_b1, radial_w2):
    grid = E // _EB
    full = lambda i: (0, 0)
    return pl.pallas_call(
        _radial_body,
        grid=(grid,),
        in_specs=[
            pl.BlockSpec((_EB, 1), lambda i: (i, 0)),
            pl.BlockSpec((NB, 64), full),
            pl.BlockSpec((1, 64), full),
            pl.BlockSpec((64, H), full),
            pl.BlockSpec((NB, 64), full),
            pl.BlockSpec((1, 64), full),
            pl.BlockSpec((64, H), full),
        ],
        out_specs=[
            pl.BlockSpec((_EB, H), lambda i: (i, 0)),
            pl.BlockSpec((_EB, H), lambda i: (i, 0)),
        ],
        out_shape=[
            jax.ShapeDtypeStruct((E, H), jnp.float32),
            jax.ShapeDtypeStruct((E, H), jnp.float32),
        ],
    )(d2.reshape(E, 1), radial_w1[0], radial_b1[0:1, :], radial_w2[0],
      radial_w1[1], radial_b1[1:2, :], radial_w2[1])


def _embed_body(z_ref, we_ref, h_ref):
    z = z_ref[...]
    oh = (z == lax.broadcasted_iota(jnp.int32, (N, NUM_EL), 1)).astype(jnp.float32)
    h_ref[...] = jnp.dot(oh, we_ref[...], preferred_element_type=jnp.float32)


def _embed(atomic_numbers, W_embed):
    return pl.pallas_call(
        _embed_body,
        out_shape=jax.ShapeDtypeStruct((N, H), jnp.float32),
    )(atomic_numbers.astype(jnp.int32).reshape(N, 1), W_embed)


def _update_body(a0_ref, a1_ref, h_ref, wm_ref, wr_ref, hn_ref, e_ref):
    agg = a0_ref[...] + a1_ref[...]
    hn = jax.nn.silu(jnp.dot(agg, wm_ref[...], preferred_element_type=jnp.float32)) + h_ref[...]
    hn_ref[...] = hn
    e_ref[...] = jnp.sum(hn * wr_ref[...])[None, None]


def _update(aggp, h, Wm, Wr):
    return pl.pallas_call(
        _update_body,
        out_shape=[
            jax.ShapeDtypeStruct((N, H), jnp.float32),
            jax.ShapeDtypeStruct((1, 1), jnp.float32),
        ],
    )(aggp[0], aggp[1], h, Wm, Wr.reshape(1, H))


def kernel(positions, atomic_numbers, pairs, shifts, W_embed, radial_w1,
           radial_b1, radial_w2, W_msg, W_read):
    src = jnp.copy(pairs[:, 0].astype(jnp.int32))
    dst = jnp.copy(pairs[:, 1].astype(jnp.int32))
    sx = jnp.copy(shifts[:, 0])
    sy = jnp.copy(shifts[:, 1])
    sz = jnp.copy(shifts[:, 2])
    px = jnp.copy(positions[:, 0])
    py = jnp.copy(positions[:, 1])
    pz = jnp.copy(positions[:, 2])

    d2 = _d2_kernel(src, dst, sx, sy, sz, px, py, pz)
    rt0, rt1 = _radial(d2, radial_w1, radial_b1, radial_w2)
    h = _embed(atomic_numbers, W_embed)
    zero = jnp.zeros((N, H), jnp.float32)

    energy = jnp.zeros((), jnp.float32)
    for t, rt in enumerate((rt0, rt1)):
        aggp = _layer_kernel(h, rt, src, dst, zero)
        h, e = _update(aggp, h, W_msg[t], W_read[t])
        energy = energy + e[0, 0]
    return energy.reshape(1)


# speedup vs baseline: 7.7441x; 1.2933x over previous
import functools

import jax
import jax.numpy as jnp
from jax import lax
from jax.experimental import pallas as pl
from jax.experimental.pallas import tpu as pltpu
from jax.experimental.pallas import tpu_sc as plsc

N = 10000
E = 320000
NUM_EL = 4
H = 128
NB = 8
L = 2
R_MAX = 5.0
LENGTH_CONV = 10.0

NC = 2
NS = 16
NW = NC * NS
EPW = E // NW
CHUNK = 80
NCHUNK = EPW // CHUNK
RPW = 624

_SC_MESH = plsc.VectorSubcoreMesh(core_axis_name="c", subcore_axis_name="s")


def _d2_body(src_hbm, dst_hbm, sx_hbm, sy_hbm, sz_hbm,
             px_hbm, py_hbm, pz_hbm, d2_hbm,
             px_v, py_v, pz_v, src_v, dst_v, sx_v, sy_v, sz_v, d2_v):
    c = lax.axis_index("c")
    s = lax.axis_index("s")
    wid = s * NC + c
    base = wid * EPW
    pltpu.sync_copy(px_hbm, px_v)
    pltpu.sync_copy(py_hbm, py_v)
    pltpu.sync_copy(pz_hbm, pz_v)
    pltpu.sync_copy(src_hbm.at[pl.ds(base, EPW)], src_v)
    pltpu.sync_copy(dst_hbm.at[pl.ds(base, EPW)], dst_v)
    pltpu.sync_copy(sx_hbm.at[pl.ds(base, EPW)], sx_v)
    pltpu.sync_copy(sy_hbm.at[pl.ds(base, EPW)], sy_v)
    pltpu.sync_copy(sz_hbm.at[pl.ds(base, EPW)], sz_v)

    def step(i, carry):
        sl = pl.ds(i * 16, 16)
        vs = src_v[sl]
        vd = dst_v[sl]
        dx = plsc.load_gather(px_v, [vd]) - plsc.load_gather(px_v, [vs]) - sx_v[sl]
        dy = plsc.load_gather(py_v, [vd]) - plsc.load_gather(py_v, [vs]) - sy_v[sl]
        dz = plsc.load_gather(pz_v, [vd]) - plsc.load_gather(pz_v, [vs]) - sz_v[sl]
        d2_v[sl] = (dx * dx + dy * dy + dz * dz) * (LENGTH_CONV * LENGTH_CONV)
        return carry

    lax.fori_loop(0, EPW // 16, step, 0)
    pltpu.sync_copy(d2_v, d2_hbm.at[pl.ds(base, EPW)])


@functools.partial(
    pl.kernel,
    out_type=jax.ShapeDtypeStruct((E,), jnp.float32),
    mesh=_SC_MESH,
    scratch_types=[
        pltpu.VMEM((N,), jnp.float32),
        pltpu.VMEM((N,), jnp.float32),
        pltpu.VMEM((N,), jnp.float32),
        pltpu.VMEM((EPW,), jnp.int32),
        pltpu.VMEM((EPW,), jnp.int32),
        pltpu.VMEM((EPW,), jnp.float32),
        pltpu.VMEM((EPW,), jnp.float32),
        pltpu.VMEM((EPW,), jnp.float32),
        pltpu.VMEM((EPW,), jnp.float32),
    ],
    compiler_params=pltpu.CompilerParams(needs_layout_passes=False),
)
def _d2_kernel(*refs):
    _d2_body(*refs)


def _layer_body(h_hbm, rt_hbm, src_hbm, dst_hbm, zero_hbm, out_hbm,
                src_v, dst_v, rt_v, hs_v, hd_v, agg_sh, sem):
    c = lax.axis_index("c")
    s = lax.axis_index("s")
    wid = s * NC + c
    pltpu.sync_copy(zero_hbm.at[pl.ds(s * RPW, RPW)], agg_sh.at[pl.ds(s * RPW, RPW)])

    @pl.when(s == NS - 1)
    def _():
        pltpu.sync_copy(zero_hbm.at[pl.ds(NS * RPW, N - NS * RPW)],
                        agg_sh.at[pl.ds(NS * RPW, N - NS * RPW)])

    plsc.subcore_barrier()

    base0 = wid * EPW

    def chunk(i, carry):
        base = base0 + i * CHUNK
        pltpu.sync_copy(src_hbm.at[pl.ds(base, CHUNK)], src_v)
        pltpu.sync_copy(dst_hbm.at[pl.ds(base, CHUNK)], dst_v)
        pltpu.sync_copy(rt_hbm.at[pl.ds(base, CHUNK)], rt_v)
        cp1 = pltpu.async_copy(h_hbm.at[src_v], hs_v, sem)
        cp2 = pltpu.async_copy(h_hbm.at[dst_v], hd_v, sem)
        cp1.wait()
        cp2.wait()

        def row(r, rcarry):
            for cc in range(H // 16):
                fsl = pl.ds(cc * 16, 16)
                rt = rt_v[r, fsl]
                hs_v[r, fsl] = hs_v[r, fsl] * rt
                hd_v[r, fsl] = hd_v[r, fsl] * rt
            return rcarry

        lax.fori_loop(0, CHUNK, row, 0)
        pltpu.sync_copy(hs_v, agg_sh.at[dst_v], add=True)
        pltpu.sync_copy(hd_v, agg_sh.at[src_v], add=True)
        return carry

    lax.fori_loop(0, NCHUNK, chunk, 0)
    plsc.subcore_barrier()
    pltpu.sync_copy(agg_sh.at[pl.ds(s * RPW, RPW)], out_hbm.at[c, pl.ds(s * RPW, RPW)])

    @pl.when(s == NS - 1)
    def _():
        pltpu.sync_copy(agg_sh.at[pl.ds(NS * RPW, N - NS * RPW)],
                        out_hbm.at[c, pl.ds(NS * RPW, N - NS * RPW)])


@functools.partial(
    pl.kernel,
    out_type=jax.ShapeDtypeStruct((NC, N, H), jnp.float32),
    mesh=_SC_MESH,
    scratch_types=[
        pltpu.VMEM((CHUNK,), jnp.int32),
        pltpu.VMEM((CHUNK,), jnp.int32),
        pltpu.VMEM((CHUNK, H), jnp.float32),
        pltpu.VMEM((CHUNK, H), jnp.float32),
        pltpu.VMEM((CHUNK, H), jnp.float32),
        pltpu.VMEM_SHARED((N, H), jnp.float32),
        pltpu.SemaphoreType.DMA,
    ],
    compiler_params=pltpu.CompilerParams(needs_layout_passes=False),
)
def _layer_kernel(*refs):
    _layer_body(*refs)


_SBR = 400
_GR = E // 16


def _rbf_body(d2_ref, rbf_ref):
    d2 = d2_ref[...]
    rinv = lax.rsqrt(d2 + 1e-12)
    r = d2 * rinv
    u = r * (1.0 / R_MAX)
    u2 = u * u
    u3 = u2 * u
    u6 = u3 * u3
    f = 1.0 - 28.0 * u6 + 48.0 * u6 * u - 21.0 * u6 * u2
    cut = jnp.where(u < 1.0, f, 0.0)
    pref = jnp.sqrt(2.0 / R_MAX) * cut * rinv
    n_lane = (lax.broadcasted_iota(jnp.int32, (_SBR, H), 1) % NB + 1).astype(jnp.float32)
    rbf_ref[...] = jnp.sin(n_lane * (jnp.pi / R_MAX) * r) * pref


def _rbf(d2):
    d2rep = jnp.broadcast_to(d2[:, None], (E, NB)).reshape(_GR, H)
    return pl.pallas_call(
        _rbf_body,
        grid=(_GR // _SBR,),
        in_specs=[pl.BlockSpec((_SBR, H), lambda i: (i, 0))],
        out_specs=pl.BlockSpec((_SBR, H), lambda i: (i, 0)),
        out_shape=jax.ShapeDtypeStruct((_GR, H), jnp.float32),
    )(d2rep)


_EB = 4000


def _radial_mlp_body(rbf_ref, w1c_ref, b1c_ref, w2d_ref, rt0_ref, rt1_ref):
    rbf = rbf_ref[...]
    phi = jax.nn.silu(jnp.dot(rbf, w1c_ref[...], preferred_element_type=jnp.float32) + b1c_ref[...])
    rt = jnp.dot(phi, w2d_ref[...], preferred_element_type=jnp.float32)
    rt0_ref[...] = rt[:, :H]
    rt1_ref[...] = rt[:, H:]


def _radial(d2, radial_w1, radial_b1, radial_w2):
    rbf = _rbf(d2).reshape(E, NB)
    w1c = jnp.concatenate([radial_w1[0], radial_w1[1]], axis=1)
    b1c = jnp.concatenate([radial_b1[0], radial_b1[1]], axis=0).reshape(1, 2 * 64)
    w2d = jnp.zeros((H, 2 * H), jnp.float32)
    w2d = w2d.at[:64, :H].set(radial_w2[0]).at[64:, H:].set(radial_w2[1])
    full = lambda i: (0, 0)
    return pl.pallas_call(
        _radial_mlp_body,
        grid=(E // _EB,),
        in_specs=[
            pl.BlockSpec((_EB, NB), lambda i: (i, 0)),
            pl.BlockSpec((NB, 2 * 64), full),
            pl.BlockSpec((1, 2 * 64), full),
            pl.BlockSpec((H, 2 * H), full),
        ],
        out_specs=[
            pl.BlockSpec((_EB, H), lambda i: (i, 0)),
            pl.BlockSpec((_EB, H), lambda i: (i, 0)),
        ],
        out_shape=[
            jax.ShapeDtypeStruct((E, H), jnp.float32),
            jax.ShapeDtypeStruct((E, H), jnp.float32),
        ],
    )(rbf, w1c, b1c, w2d)


def _embed_body(z_ref, we_ref, h_ref):
    z = z_ref[...]
    oh = (z == lax.broadcasted_iota(jnp.int32, (N, NUM_EL), 1)).astype(jnp.float32)
    h_ref[...] = jnp.dot(oh, we_ref[...], preferred_element_type=jnp.float32)


def _embed(atomic_numbers, W_embed):
    return pl.pallas_call(
        _embed_body,
        out_shape=jax.ShapeDtypeStruct((N, H), jnp.float32),
    )(atomic_numbers.astype(jnp.int32).reshape(N, 1), W_embed)


def _update_body(a0_ref, a1_ref, h_ref, wm_ref, wr_ref, hn_ref, e_ref):
    agg = a0_ref[...] + a1_ref[...]
    hn = jax.nn.silu(jnp.dot(agg, wm_ref[...], preferred_element_type=jnp.float32)) + h_ref[...]
    hn_ref[...] = hn
    e_ref[...] = jnp.sum(hn * wr_ref[...])[None, None]


def _update(aggp, h, Wm, Wr):
    return pl.pallas_call(
        _update_body,
        out_shape=[
            jax.ShapeDtypeStruct((N, H), jnp.float32),
            jax.ShapeDtypeStruct((1, 1), jnp.float32),
        ],
    )(aggp[0], aggp[1], h, Wm, Wr.reshape(1, H))


def kernel(positions, atomic_numbers, pairs, shifts, W_embed, radial_w1,
           radial_b1, radial_w2, W_msg, W_read):
    src = jnp.copy(pairs[:, 0].astype(jnp.int32))
    dst = jnp.copy(pairs[:, 1].astype(jnp.int32))
    sx = jnp.copy(shifts[:, 0])
    sy = jnp.copy(shifts[:, 1])
    sz = jnp.copy(shifts[:, 2])
    px = jnp.copy(positions[:, 0])
    py = jnp.copy(positions[:, 1])
    pz = jnp.copy(positions[:, 2])

    d2 = _d2_kernel(src, dst, sx, sy, sz, px, py, pz)
    rt0, rt1 = _radial(d2, radial_w1, radial_b1, radial_w2)
    h = _embed(atomic_numbers, W_embed)
    zero = jnp.zeros((N, H), jnp.float32)

    energy = jnp.zeros((), jnp.float32)
    for t, rt in enumerate((rt0, rt1)):
        aggp = _layer_kernel(h, rt, src, dst, zero)
        h, e = _update(aggp, h, W_msg[t], W_read[t])
        energy = energy + e[0, 0]
    return energy.reshape(1)
